# baseline (speedup 1.0000x reference)
import functools

import jax
import jax.numpy as jnp
from jax import lax
from jax.experimental import pallas as pl
from jax.experimental.pallas import tpu as pltpu
from jax.experimental.pallas import tpu_sc as plsc

NW = 32
NC = 2
NS = 16
C = 80


def _sc_gather(nl, senders2, receivers2, E, N, D):
    EPW = E // NW
    CG = 128
    NCH = EPW // CG
    TAIL = EPW - NCH * CG
    mesh = plsc.VectorSubcoreMesh(core_axis_name="c", subcore_axis_name="s")

    @functools.partial(
        pl.kernel,
        out_type=(jax.ShapeDtypeStruct((E, D), jnp.int32),
                  jax.ShapeDtypeStruct((E, D), jnp.int32)),
        mesh=mesh,
        scratch_types=[
            pltpu.VMEM((EPW,), jnp.int32),
            pltpu.VMEM((EPW,), jnp.int32),
            pltpu.VMEM((2, CG, D), jnp.int32),
            pltpu.VMEM((2, CG, D), jnp.int32),
            pltpu.SemaphoreType.DMA,
            pltpu.SemaphoreType.DMA,
            pltpu.SemaphoreType.DMA,
            pltpu.SemaphoreType.DMA,
            pltpu.SemaphoreType.DMA,
            pltpu.SemaphoreType.DMA,
            pltpu.SemaphoreType.DMA,
            pltpu.SemaphoreType.DMA,
        ],
        compiler_params=pltpu.CompilerParams(use_tc_tiling_on_sc=False),
    )
    def k(nl_hbm, s_hbm, r_hbm, gs_hbm, gr_hbm, sidx, ridx, srow, rrow,
          sg0, sg1, rg0, rg1, sw0, sw1, rw0, rw1):
        cid = lax.axis_index("c")
        sid = lax.axis_index("s")
        wid = sid * NC + cid
        base = wid * EPW
        pltpu.sync_copy(s_hbm.at[wid], sidx)
        pltpu.sync_copy(r_hbm.at[wid], ridx)

        def fire(i, b, gsem, rsem):
            pltpu.async_copy(nl_hbm.at[sidx.at[pl.ds(i * CG, CG)]],
                             srow.at[b], gsem)
            pltpu.async_copy(nl_hbm.at[ridx.at[pl.ds(i * CG, CG)]],
                             rrow.at[b], rsem)

        def wait_gather(i, b, gsem, rsem):
            pltpu.make_async_copy(nl_hbm.at[sidx.at[pl.ds(i * CG, CG)]],
                                  srow.at[b], gsem).wait()
            pltpu.make_async_copy(nl_hbm.at[ridx.at[pl.ds(i * CG, CG)]],
                                  rrow.at[b], rsem).wait()

        def fire_write(i, b, wsem_s, wsem_r):
            off = base + i * CG
            pltpu.async_copy(srow.at[b], gs_hbm.at[pl.ds(off, CG)], wsem_s)
            pltpu.async_copy(rrow.at[b], gr_hbm.at[pl.ds(off, CG)], wsem_r)

        def wait_write(i, b, wsem_s, wsem_r):
            off = base + i * CG
            pltpu.make_async_copy(srow.at[b], gs_hbm.at[pl.ds(off, CG)],
                                  wsem_s).wait()
            pltpu.make_async_copy(rrow.at[b], gr_hbm.at[pl.ds(off, CG)],
                                  wsem_r).wait()

        fire(0, 0, sg0, rg0)
        fire(1, 1, sg1, rg1)

        def body(j, carry):
            i0 = 2 * j
            i1 = 2 * j + 1
            wait_gather(i0, 0, sg0, rg0)
            fire_write(i0, 0, sw0, rw0)
            wait_gather(i1, 1, sg1, rg1)
            fire_write(i1, 1, sw1, rw1)
            wait_write(i0, 0, sw0, rw0)

            @pl.when(i0 + 2 < NCH)
            def _():
                fire(i0 + 2, 0, sg0, rg0)

            wait_write(i1, 1, sw1, rw1)

            @pl.when(i1 + 2 < NCH)
            def _():
                fire(i1 + 2, 1, sg1, rg1)

            return carry

        lax.fori_loop(0, NCH // 2, body, 0)

        toff = NCH * CG
        pltpu.async_copy(nl_hbm.at[sidx.at[pl.ds(toff, TAIL)]],
                         srow.at[0, pl.ds(0, TAIL)], sg0)
        pltpu.async_copy(nl_hbm.at[ridx.at[pl.ds(toff, TAIL)]],
                         rrow.at[0, pl.ds(0, TAIL)], rg0)
        pltpu.make_async_copy(nl_hbm.at[sidx.at[pl.ds(toff, TAIL)]],
                              srow.at[0, pl.ds(0, TAIL)], sg0).wait()
        pltpu.make_async_copy(nl_hbm.at[ridx.at[pl.ds(toff, TAIL)]],
                              rrow.at[0, pl.ds(0, TAIL)], rg0).wait()
        pltpu.sync_copy(srow.at[0, pl.ds(0, TAIL)],
                        gs_hbm.at[pl.ds(base + toff, TAIL)])
        pltpu.sync_copy(rrow.at[0, pl.ds(0, TAIL)],
                        gr_hbm.at[pl.ds(base + toff, TAIL)])

    return k(nl, senders2, receivers2)


def _sc_scatter_add(new_edge, receivers3, zeros_nd, E, N, D):
    NCH = receivers3.shape[1]
    EPW = NCH * C
    SLAB = (N // NS) // 8 * 8
    TAIL_OFF = SLAB * NS
    TAIL = N - TAIL_OFF
    mesh = plsc.VectorSubcoreMesh(core_axis_name="c", subcore_axis_name="s")

    @functools.partial(
        pl.kernel,
        out_type=jax.ShapeDtypeStruct((NC, N, D), jnp.float32),
        mesh=mesh,
        scratch_types=[
            pltpu.VMEM((NCH, C), jnp.int32),
            pltpu.VMEM((C, D), jnp.float32),
            pltpu.VMEM_SHARED((N, D), jnp.float32),
        ],
    )
    def k(ne_hbm, r_hbm, z_hbm, out_hbm, ridx, rows, aggr_sh):
        cid = lax.axis_index("c")
        sid = lax.axis_index("s")
        wid = sid * NC + cid
        base = wid * EPW
        pltpu.sync_copy(z_hbm.at[pl.ds(sid * SLAB, SLAB)],
                        aggr_sh.at[pl.ds(sid * SLAB, SLAB)])

        @pl.when(sid == 0)
        def _():
            pltpu.sync_copy(z_hbm.at[pl.ds(TAIL_OFF, TAIL)],
                            aggr_sh.at[pl.ds(TAIL_OFF, TAIL)])

        plsc.subcore_barrier()
        pltpu.sync_copy(r_hbm.at[wid], ridx)

        def body(i, carry):
            pltpu.sync_copy(ne_hbm.at[pl.ds(base + i * C, C)], rows)
            pltpu.sync_copy(rows, aggr_sh.at[ridx.at[i]], add=True)
            return carry

        lax.fori_loop(0, NCH, body, 0)
        plsc.subcore_barrier()
        pltpu.sync_copy(aggr_sh.at[pl.ds(sid * SLAB, SLAB)],
                        out_hbm.at[cid].at[pl.ds(sid * SLAB, SLAB)])

        @pl.when(sid == 0)
        def _():
            pltpu.sync_copy(aggr_sh.at[pl.ds(TAIL_OFF, TAIL)],
                            out_hbm.at[cid].at[pl.ds(TAIL_OFF, TAIL)])

    return k(new_edge, receivers3, zeros_nd)


def _edge_mlp_body(gs_ref, gr_ref, ef_ref, w1_ref, b1_ref, w2_ref, b2_ref,
                   g_ref, bg_ref, ne_ref, eo_ref):
    ef = ef_ref[...]
    ws = gs_ref[...]
    wr = gr_ref[...]
    mask = jnp.int32(-65536)
    gse = jax.lax.bitcast_convert_type(ws << 16, jnp.float32)
    gso = jax.lax.bitcast_convert_type(ws & mask, jnp.float32)
    gre = jax.lax.bitcast_convert_type(wr << 16, jnp.float32)
    gro = jax.lax.bitcast_convert_type(wr & mask, jnp.float32)
    x = jnp.concatenate([gse, gso, gre, gro, ef], axis=-1)
    h = jnp.dot(x, w1_ref[...], preferred_element_type=jnp.float32)
    h = jnp.maximum(h + b1_ref[...], 0.0)
    h = jnp.dot(h, w2_ref[...], preferred_element_type=jnp.float32)
    h = jnp.maximum(h + b2_ref[...], 0.0)
    mu = jnp.mean(h, axis=-1, keepdims=True)
    var = jnp.mean((h - mu) ** 2, axis=-1, keepdims=True)
    ne = (h - mu) / jnp.sqrt(var + 1e-5) * g_ref[...] + bg_ref[...]
    ne_ref[...] = ne
    eo_ref[...] = ef + ne


def _tc_edge_mlp(gs, gr, ef, We1, be1, We2, be2, ge, bge, E, D, BE=2000):
    grid = (E // BE,)
    blk = pl.BlockSpec((BE, D), lambda i: (i, 0))
    blkh = pl.BlockSpec((BE, D // 2), lambda i: (i, 0))
    full = lambda a: pl.BlockSpec(a.shape, lambda i: tuple(0 for _ in a.shape))
    return pl.pallas_call(
        _edge_mlp_body,
        grid=grid,
        in_specs=[blkh, blkh, blk, full(We1), full(be1), full(We2), full(be2),
                  full(ge), full(bge)],
        out_specs=[blk, blk],
        out_shape=[jax.ShapeDtypeStruct((E, D), jnp.float32),
                   jax.ShapeDtypeStruct((E, D), jnp.float32)],
        compiler_params=pltpu.CompilerParams(
            dimension_semantics=("arbitrary",)),
    )(gs, gr, ef, We1, be1, We2, be2, ge, bge)


def _node_mlp_body(nl_ref, a0_ref, a1_ref, w1_ref, b1_ref, w2_ref, b2_ref,
                   g_ref, bg_ref, out_ref):
    nl = nl_ref[...]
    aggr = a0_ref[...] + a1_ref[...]
    x = jnp.concatenate([nl, aggr], axis=-1)
    h = jnp.dot(x, w1_ref[...], preferred_element_type=jnp.float32)
    h = jnp.maximum(h + b1_ref[...], 0.0)
    h = jnp.dot(h, w2_ref[...], preferred_element_type=jnp.float32)
    h = jnp.maximum(h + b2_ref[...], 0.0)
    mu = jnp.mean(h, axis=-1, keepdims=True)
    var = jnp.mean((h - mu) ** 2, axis=-1, keepdims=True)
    nn = (h - mu) / jnp.sqrt(var + 1e-5) * g_ref[...] + bg_ref[...]
    out_ref[...] = nn + nl


def _tc_node_mlp(nl, aggr2, Wn1, bn1, Wn2, bn2, gn, bgn, N, D, BN=2000):
    grid = (N // BN,)
    blk = pl.BlockSpec((BN, D), lambda i: (i, 0))
    full = lambda a: pl.BlockSpec(a.shape, lambda i: tuple(0 for _ in a.shape))
    return pl.pallas_call(
        _node_mlp_body,
        grid=grid,
        in_specs=[blk, blk, blk, full(Wn1), full(bn1), full(Wn2), full(bn2),
                  full(gn), full(bgn)],
        out_specs=blk,
        out_shape=jax.ShapeDtypeStruct((N, D), jnp.float32),
        compiler_params=pltpu.CompilerParams(
            dimension_semantics=("arbitrary",)),
    )(nl, aggr2[0], aggr2[1], Wn1, bn1, Wn2, bn2, gn, bgn)


def kernel(node_latents, edge_features, senders, receivers, We1, be1, We2,
           be2, ge, bge, Wn1, bn1, Wn2, bn2, gn, bgn):
    B, N, D = node_latents.shape
    E = senders.shape[0]
    EPW = E // NW
    NCH = EPW // C

    nl = node_latents.reshape(N, D)
    ef = edge_features.reshape(E, D)
    s2 = senders.astype(jnp.int32).reshape(NW, EPW)
    r2 = receivers.astype(jnp.int32).reshape(NW, EPW)
    r3 = receivers.astype(jnp.int32).reshape(NW, NCH, C)

    nl32 = jax.lax.bitcast_convert_type(
        nl.astype(jnp.bfloat16).reshape(N, D // 2, 2), jnp.int32)
    gs, gr = _sc_gather(nl32, s2, r2, E, N, D // 2)
    W1s, W1r, W1e = We1[:D], We1[D:2 * D], We1[2 * D:]
    We1p = jnp.concatenate([W1s[0::2], W1s[1::2], W1r[0::2], W1r[1::2], W1e],
                           axis=0)
    new_edge, edge_out = _tc_edge_mlp(
        gs, gr, ef, We1p, be1.reshape(1, D), We2, be2.reshape(1, D),
        ge.reshape(1, D), bge.reshape(1, D), E, D)
    zeros_nd = jnp.zeros((N, D), jnp.float32)
    aggr2 = _sc_scatter_add(new_edge, r3, zeros_nd, E, N, D)
    node_out = _tc_node_mlp(
        nl, aggr2, Wn1, bn1.reshape(1, D), Wn2, bn2.reshape(1, D),
        gn.reshape(1, D), bgn.reshape(1, D), N, D)
    return node_out.reshape(B, N, D), edge_out.reshape(B, E, D)


# speedup vs baseline: 1.0476x; 1.0476x over previous
import functools

import jax
import jax.numpy as jnp
from jax import lax
from jax.experimental import pallas as pl
from jax.experimental.pallas import tpu as pltpu
from jax.experimental.pallas import tpu_sc as plsc

NW = 32
NC = 2
NS = 16
C = 80


def _sc_gather(nl, senders2, receivers2, E, N, D):
    EPW = E // NW
    CG = 128
    NCH = EPW // CG
    TAIL = EPW - NCH * CG
    mesh = plsc.VectorSubcoreMesh(core_axis_name="c", subcore_axis_name="s")

    @functools.partial(
        pl.kernel,
        out_type=(jax.ShapeDtypeStruct((E // 2, 2 * D), jnp.int32),
                  jax.ShapeDtypeStruct((E // 2, 2 * D), jnp.int32)),
        mesh=mesh,
        scratch_types=[
            pltpu.VMEM((EPW,), jnp.int32),
            pltpu.VMEM((EPW,), jnp.int32),
            pltpu.VMEM((2, CG, D), jnp.int32),
            pltpu.VMEM((2, CG, D), jnp.int32),
            pltpu.SemaphoreType.DMA,
            pltpu.SemaphoreType.DMA,
            pltpu.SemaphoreType.DMA,
            pltpu.SemaphoreType.DMA,
            pltpu.SemaphoreType.DMA,
            pltpu.SemaphoreType.DMA,
            pltpu.SemaphoreType.DMA,
            pltpu.SemaphoreType.DMA,
        ],
        compiler_params=pltpu.CompilerParams(use_tc_tiling_on_sc=False),
    )
    def k(nl_hbm, s_hbm, r_hbm, gs_hbm, gr_hbm, sidx, ridx, srow, rrow,
          sg0, sg1, rg0, rg1, sw0, sw1, rw0, rw1):
        cid = lax.axis_index("c")
        sid = lax.axis_index("s")
        wid = sid * NC + cid
        base = wid * EPW
        first_half = wid < NW // 2
        rowbase = base - jnp.where(first_half, 0, E // 2)
        colbase = jnp.where(first_half, 0, D)
        pltpu.sync_copy(s_hbm.at[wid], sidx)
        pltpu.sync_copy(r_hbm.at[wid], ridx)

        def fire(i, b, gsem, rsem):
            pltpu.async_copy(nl_hbm.at[sidx.at[pl.ds(i * CG, CG)]],
                             srow.at[b], gsem)
            pltpu.async_copy(nl_hbm.at[ridx.at[pl.ds(i * CG, CG)]],
                             rrow.at[b], rsem)

        def wait_gather(i, b, gsem, rsem):
            pltpu.make_async_copy(nl_hbm.at[sidx.at[pl.ds(i * CG, CG)]],
                                  srow.at[b], gsem).wait()
            pltpu.make_async_copy(nl_hbm.at[ridx.at[pl.ds(i * CG, CG)]],
                                  rrow.at[b], rsem).wait()

        def fire_write(i, b, wsem_s, wsem_r):
            off = rowbase + i * CG
            pltpu.async_copy(srow.at[b],
                             gs_hbm.at[pl.ds(off, CG), pl.ds(colbase, D)],
                             wsem_s)
            pltpu.async_copy(rrow.at[b],
                             gr_hbm.at[pl.ds(off, CG), pl.ds(colbase, D)],
                             wsem_r)

        def wait_write(i, b, wsem_s, wsem_r):
            off = rowbase + i * CG
            pltpu.make_async_copy(srow.at[b],
                                  gs_hbm.at[pl.ds(off, CG),
                                            pl.ds(colbase, D)],
                                  wsem_s).wait()
            pltpu.make_async_copy(rrow.at[b],
                                  gr_hbm.at[pl.ds(off, CG),
                                            pl.ds(colbase, D)],
                                  wsem_r).wait()

        fire(0, 0, sg0, rg0)
        fire(1, 1, sg1, rg1)

        def body(j, carry):
            i0 = 2 * j
            i1 = 2 * j + 1
            wait_gather(i0, 0, sg0, rg0)
            fire_write(i0, 0, sw0, rw0)
            wait_gather(i1, 1, sg1, rg1)
            fire_write(i1, 1, sw1, rw1)
            wait_write(i0, 0, sw0, rw0)

            @pl.when(i0 + 2 < NCH)
            def _():
                fire(i0 + 2, 0, sg0, rg0)

            wait_write(i1, 1, sw1, rw1)

            @pl.when(i1 + 2 < NCH)
            def _():
                fire(i1 + 2, 1, sg1, rg1)

            return carry

        lax.fori_loop(0, NCH // 2, body, 0)

        toff = NCH * CG
        pltpu.async_copy(nl_hbm.at[sidx.at[pl.ds(toff, TAIL)]],
                         srow.at[0, pl.ds(0, TAIL)], sg0)
        pltpu.async_copy(nl_hbm.at[ridx.at[pl.ds(toff, TAIL)]],
                         rrow.at[0, pl.ds(0, TAIL)], rg0)
        pltpu.make_async_copy(nl_hbm.at[sidx.at[pl.ds(toff, TAIL)]],
                              srow.at[0, pl.ds(0, TAIL)], sg0).wait()
        pltpu.make_async_copy(nl_hbm.at[ridx.at[pl.ds(toff, TAIL)]],
                              rrow.at[0, pl.ds(0, TAIL)], rg0).wait()
        pltpu.sync_copy(srow.at[0, pl.ds(0, TAIL)],
                        gs_hbm.at[pl.ds(rowbase + toff, TAIL),
                                  pl.ds(colbase, D)])
        pltpu.sync_copy(rrow.at[0, pl.ds(0, TAIL)],
                        gr_hbm.at[pl.ds(rowbase + toff, TAIL),
                                  pl.ds(colbase, D)])

    return k(nl, senders2, receivers2)


def _sc_scatter_add(new_edge, receivers3, zeros_nd, E, N, D):
    NCH = receivers3.shape[1]
    EPW = NCH * C
    SLAB = (N // NS) // 8 * 8
    TAIL_OFF = SLAB * NS
    TAIL = N - TAIL_OFF
    mesh = plsc.VectorSubcoreMesh(core_axis_name="c", subcore_axis_name="s")

    @functools.partial(
        pl.kernel,
        out_type=jax.ShapeDtypeStruct((NC, N, D), jnp.float32),
        mesh=mesh,
        scratch_types=[
            pltpu.VMEM((NCH, C), jnp.int32),
            pltpu.VMEM((C, D), jnp.float32),
            pltpu.VMEM_SHARED((N, D), jnp.float32),
        ],
    )
    def k(ne_hbm, r_hbm, z_hbm, out_hbm, ridx, rows, aggr_sh):
        cid = lax.axis_index("c")
        sid = lax.axis_index("s")
        wid = sid * NC + cid
        base = wid * EPW
        pltpu.sync_copy(z_hbm.at[pl.ds(sid * SLAB, SLAB)],
                        aggr_sh.at[pl.ds(sid * SLAB, SLAB)])

        @pl.when(sid == 0)
        def _():
            pltpu.sync_copy(z_hbm.at[pl.ds(TAIL_OFF, TAIL)],
                            aggr_sh.at[pl.ds(TAIL_OFF, TAIL)])

        plsc.subcore_barrier()
        pltpu.sync_copy(r_hbm.at[wid], ridx)

        def body(i, carry):
            pltpu.sync_copy(ne_hbm.at[pl.ds(base + i * C, C)], rows)
            pltpu.sync_copy(rows, aggr_sh.at[ridx.at[i]], add=True)
            return carry

        lax.fori_loop(0, NCH, body, 0)
        plsc.subcore_barrier()
        pltpu.sync_copy(aggr_sh.at[pl.ds(sid * SLAB, SLAB)],
                        out_hbm.at[cid].at[pl.ds(sid * SLAB, SLAB)])

        @pl.when(sid == 0)
        def _():
            pltpu.sync_copy(aggr_sh.at[pl.ds(TAIL_OFF, TAIL)],
                            out_hbm.at[cid].at[pl.ds(TAIL_OFF, TAIL)])

    return k(new_edge, receivers3, zeros_nd)


def _edge_mlp_body(gs_ref, gr_ref, ef_ref, w1_ref, b1_ref, w2_ref, b2_ref,
                   g_ref, bg_ref, ne_ref, eo_ref):
    ef = ef_ref[...]
    h = pl.program_id(1)
    ws_full = gs_ref[...]
    wr_full = gr_ref[...]
    hw = ws_full.shape[1] // 2
    ws = jnp.where(h == 0, ws_full[:, :hw], ws_full[:, hw:])
    wr = jnp.where(h == 0, wr_full[:, :hw], wr_full[:, hw:])
    mask = jnp.int32(-65536)
    gse = jax.lax.bitcast_convert_type(ws << 16, jnp.float32)
    gso = jax.lax.bitcast_convert_type(ws & mask, jnp.float32)
    gre = jax.lax.bitcast_convert_type(wr << 16, jnp.float32)
    gro = jax.lax.bitcast_convert_type(wr & mask, jnp.float32)
    x = jnp.concatenate([gse, gso, gre, gro, ef], axis=-1)
    hdd = jnp.dot(x, w1_ref[...], preferred_element_type=jnp.float32)
    hdd = jnp.maximum(hdd + b1_ref[...], 0.0)
    hdd = jnp.dot(hdd, w2_ref[...], preferred_element_type=jnp.float32)
    hdd = jnp.maximum(hdd + b2_ref[...], 0.0)
    mu = jnp.mean(hdd, axis=-1, keepdims=True)
    var = jnp.mean((hdd - mu) ** 2, axis=-1, keepdims=True)
    ne = (hdd - mu) / jnp.sqrt(var + 1e-5) * g_ref[...] + bg_ref[...]
    ne_ref[...] = ne
    eo_ref[...] = ef + ne


def _tc_edge_mlp(gs, gr, ef, We1, be1, We2, be2, ge, bge, E, D, BE=1000):
    NBh = (E // 2) // BE
    grid = (NBh, 2)
    pair = pl.BlockSpec((BE, D), lambda j, h: (j, 0))
    ehalf = pl.BlockSpec((BE, D), lambda j, h: (h * NBh + j, 0))
    full = lambda a: pl.BlockSpec(a.shape, lambda j, h: tuple(0 for _ in a.shape))
    return pl.pallas_call(
        _edge_mlp_body,
        grid=grid,
        in_specs=[pair, pair, ehalf, full(We1), full(be1), full(We2),
                  full(be2), full(ge), full(bge)],
        out_specs=[ehalf, ehalf],
        out_shape=[jax.ShapeDtypeStruct((E, D), jnp.float32),
                   jax.ShapeDtypeStruct((E, D), jnp.float32)],
        compiler_params=pltpu.CompilerParams(
            dimension_semantics=("arbitrary", "arbitrary")),
    )(gs, gr, ef, We1, be1, We2, be2, ge, bge)


def _node_mlp_body(nl_ref, a0_ref, a1_ref, w1_ref, b1_ref, w2_ref, b2_ref,
                   g_ref, bg_ref, out_ref):
    nl = nl_ref[...]
    aggr = a0_ref[...] + a1_ref[...]
    x = jnp.concatenate([nl, aggr], axis=-1)
    h = jnp.dot(x, w1_ref[...], preferred_element_type=jnp.float32)
    h = jnp.maximum(h + b1_ref[...], 0.0)
    h = jnp.dot(h, w2_ref[...], preferred_element_type=jnp.float32)
    h = jnp.maximum(h + b2_ref[...], 0.0)
    mu = jnp.mean(h, axis=-1, keepdims=True)
    var = jnp.mean((h - mu) ** 2, axis=-1, keepdims=True)
    nn = (h - mu) / jnp.sqrt(var + 1e-5) * g_ref[...] + bg_ref[...]
    out_ref[...] = nn + nl


def _tc_node_mlp(nl, aggr2, Wn1, bn1, Wn2, bn2, gn, bgn, N, D, BN=2000):
    grid = (N // BN,)
    blk = pl.BlockSpec((BN, D), lambda i: (i, 0))
    full = lambda a: pl.BlockSpec(a.shape, lambda i: tuple(0 for _ in a.shape))
    return pl.pallas_call(
        _node_mlp_body,
        grid=grid,
        in_specs=[blk, blk, blk, full(Wn1), full(bn1), full(Wn2), full(bn2),
                  full(gn), full(bgn)],
        out_specs=blk,
        out_shape=jax.ShapeDtypeStruct((N, D), jnp.float32),
        compiler_params=pltpu.CompilerParams(
            dimension_semantics=("arbitrary",)),
    )(nl, aggr2[0], aggr2[1], Wn1, bn1, Wn2, bn2, gn, bgn)


def kernel(node_latents, edge_features, senders, receivers, We1, be1, We2,
           be2, ge, bge, Wn1, bn1, Wn2, bn2, gn, bgn):
    B, N, D = node_latents.shape
    E = senders.shape[0]
    EPW = E // NW
    NCH = EPW // C

    nl = node_latents.reshape(N, D)
    ef = edge_features.reshape(E, D)
    s2 = senders.astype(jnp.int32).reshape(NW, EPW)
    r2 = receivers.astype(jnp.int32).reshape(NW, EPW)
    r3 = receivers.astype(jnp.int32).reshape(NW, NCH, C)

    nl32 = jax.lax.bitcast_convert_type(
        nl.astype(jnp.bfloat16).reshape(N, D // 2, 2), jnp.int32)
    gs, gr = _sc_gather(nl32, s2, r2, E, N, D // 2)
    W1s, W1r, W1e = We1[:D], We1[D:2 * D], We1[2 * D:]
    We1p = jnp.concatenate([W1s[0::2], W1s[1::2], W1r[0::2], W1r[1::2], W1e],
                           axis=0)
    new_edge, edge_out = _tc_edge_mlp(
        gs, gr, ef, We1p, be1.reshape(1, D), We2, be2.reshape(1, D),
        ge.reshape(1, D), bge.reshape(1, D), E, D)
    zeros_nd = jnp.zeros((N, D), jnp.float32)
    aggr2 = _sc_scatter_add(new_edge, r3, zeros_nd, E, N, D)
    node_out = _tc_node_mlp(
        nl, aggr2, Wn1, bn1.reshape(1, D), Wn2, bn2.reshape(1, D),
        gn.reshape(1, D), bgn.reshape(1, D), N, D)
    return node_out.reshape(B, N, D), edge_out.reshape(B, E, D)


# speedup vs baseline: 1.2263x; 1.1705x over previous
import functools

import jax
import jax.numpy as jnp
from jax import lax
from jax.experimental import pallas as pl
from jax.experimental.pallas import tpu as pltpu
from jax.experimental.pallas import tpu_sc as plsc

NW = 32
NC = 2
NS = 16
C = 80


def _sc_gather(nl, senders2, receivers2, E, N, D):
    EPW = E // NW
    CG = 128
    NCH = EPW // CG
    TAIL = EPW - NCH * CG
    mesh = plsc.VectorSubcoreMesh(core_axis_name="c", subcore_axis_name="s")

    @functools.partial(
        pl.kernel,
        out_type=(jax.ShapeDtypeStruct((E // 2, 2 * D), jnp.int32),
                  jax.ShapeDtypeStruct((E // 2, 2 * D), jnp.int32)),
        mesh=mesh,
        scratch_types=[
            pltpu.VMEM((EPW,), jnp.int32),
            pltpu.VMEM((EPW,), jnp.int32),
            pltpu.VMEM((2, CG, D), jnp.int32),
            pltpu.VMEM((2, CG, D), jnp.int32),
            pltpu.SemaphoreType.DMA,
            pltpu.SemaphoreType.DMA,
            pltpu.SemaphoreType.DMA,
            pltpu.SemaphoreType.DMA,
            pltpu.SemaphoreType.DMA,
            pltpu.SemaphoreType.DMA,
            pltpu.SemaphoreType.DMA,
            pltpu.SemaphoreType.DMA,
        ],
        compiler_params=pltpu.CompilerParams(use_tc_tiling_on_sc=False),
    )
    def k(nl_hbm, s_hbm, r_hbm, gs_hbm, gr_hbm, sidx, ridx, srow, rrow,
          sg0, sg1, rg0, rg1, sw0, sw1, rw0, rw1):
        cid = lax.axis_index("c")
        sid = lax.axis_index("s")
        wid = sid * NC + cid
        base = wid * EPW
        first_half = wid < NW // 2
        rowbase = base - jnp.where(first_half, 0, E // 2)
        colbase = jnp.where(first_half, 0, D)
        pltpu.sync_copy(s_hbm.at[wid], sidx)
        pltpu.sync_copy(r_hbm.at[wid], ridx)

        def fire(i, b, gsem, rsem):
            pltpu.async_copy(nl_hbm.at[sidx.at[pl.ds(i * CG, CG)]],
                             srow.at[b], gsem)
            pltpu.async_copy(nl_hbm.at[ridx.at[pl.ds(i * CG, CG)]],
                             rrow.at[b], rsem)

        def wait_gather(i, b, gsem, rsem):
            pltpu.make_async_copy(nl_hbm.at[sidx.at[pl.ds(i * CG, CG)]],
                                  srow.at[b], gsem).wait()
            pltpu.make_async_copy(nl_hbm.at[ridx.at[pl.ds(i * CG, CG)]],
                                  rrow.at[b], rsem).wait()

        def fire_write(i, b, wsem_s, wsem_r):
            off = rowbase + i * CG
            pltpu.async_copy(srow.at[b],
                             gs_hbm.at[pl.ds(off, CG), pl.ds(colbase, D)],
                             wsem_s)
            pltpu.async_copy(rrow.at[b],
                             gr_hbm.at[pl.ds(off, CG), pl.ds(colbase, D)],
                             wsem_r)

        def wait_write(i, b, wsem_s, wsem_r):
            off = rowbase + i * CG
            pltpu.make_async_copy(srow.at[b],
                                  gs_hbm.at[pl.ds(off, CG),
                                            pl.ds(colbase, D)],
                                  wsem_s).wait()
            pltpu.make_async_copy(rrow.at[b],
                                  gr_hbm.at[pl.ds(off, CG),
                                            pl.ds(colbase, D)],
                                  wsem_r).wait()

        fire(0, 0, sg0, rg0)
        fire(1, 1, sg1, rg1)

        def body(j, carry):
            i0 = 2 * j
            i1 = 2 * j + 1
            wait_gather(i0, 0, sg0, rg0)
            fire_write(i0, 0, sw0, rw0)
            wait_gather(i1, 1, sg1, rg1)
            fire_write(i1, 1, sw1, rw1)
            wait_write(i0, 0, sw0, rw0)

            @pl.when(i0 + 2 < NCH)
            def _():
                fire(i0 + 2, 0, sg0, rg0)

            wait_write(i1, 1, sw1, rw1)

            @pl.when(i1 + 2 < NCH)
            def _():
                fire(i1 + 2, 1, sg1, rg1)

            return carry

        lax.fori_loop(0, NCH // 2, body, 0)

        toff = NCH * CG
        pltpu.async_copy(nl_hbm.at[sidx.at[pl.ds(toff, TAIL)]],
                         srow.at[0, pl.ds(0, TAIL)], sg0)
        pltpu.async_copy(nl_hbm.at[ridx.at[pl.ds(toff, TAIL)]],
                         rrow.at[0, pl.ds(0, TAIL)], rg0)
        pltpu.make_async_copy(nl_hbm.at[sidx.at[pl.ds(toff, TAIL)]],
                              srow.at[0, pl.ds(0, TAIL)], sg0).wait()
        pltpu.make_async_copy(nl_hbm.at[ridx.at[pl.ds(toff, TAIL)]],
                              rrow.at[0, pl.ds(0, TAIL)], rg0).wait()
        pltpu.sync_copy(srow.at[0, pl.ds(0, TAIL)],
                        gs_hbm.at[pl.ds(rowbase + toff, TAIL),
                                  pl.ds(colbase, D)])
        pltpu.sync_copy(rrow.at[0, pl.ds(0, TAIL)],
                        gr_hbm.at[pl.ds(rowbase + toff, TAIL),
                                  pl.ds(colbase, D)])

    return k(nl, senders2, receivers2)


def _sc_scatter_add(new_edge, receivers3, zeros_nd, E, N, D):
    NCH = receivers3.shape[1]
    EPW = NCH * C
    SLAB = (N // NS) // 8 * 8
    TAIL_OFF = SLAB * NS
    TAIL = N - TAIL_OFF
    mesh = plsc.VectorSubcoreMesh(core_axis_name="c", subcore_axis_name="s")

    @functools.partial(
        pl.kernel,
        out_type=jax.ShapeDtypeStruct((NC, N, D), jnp.float32),
        mesh=mesh,
        scratch_types=[
            pltpu.VMEM((NCH, C), jnp.int32),
            pltpu.VMEM((C, D), jnp.float32),
            pltpu.VMEM_SHARED((N, D), jnp.float32),
        ],
    )
    def k(ne_hbm, r_hbm, z_hbm, out_hbm, ridx, rows, aggr_sh):
        cid = lax.axis_index("c")
        sid = lax.axis_index("s")
        wid = sid * NC + cid
        base = wid * EPW
        pltpu.sync_copy(z_hbm.at[pl.ds(sid * SLAB, SLAB)],
                        aggr_sh.at[pl.ds(sid * SLAB, SLAB)])

        @pl.when(sid == 0)
        def _():
            pltpu.sync_copy(z_hbm.at[pl.ds(TAIL_OFF, TAIL)],
                            aggr_sh.at[pl.ds(TAIL_OFF, TAIL)])

        plsc.subcore_barrier()
        pltpu.sync_copy(r_hbm.at[wid], ridx)

        def body(i, carry):
            pltpu.sync_copy(ne_hbm.at[pl.ds(base + i * C, C)], rows)
            pltpu.sync_copy(rows, aggr_sh.at[ridx.at[i]], add=True)
            return carry

        lax.fori_loop(0, NCH, body, 0)
        plsc.subcore_barrier()
        pltpu.sync_copy(aggr_sh.at[pl.ds(sid * SLAB, SLAB)],
                        out_hbm.at[cid].at[pl.ds(sid * SLAB, SLAB)])

        @pl.when(sid == 0)
        def _():
            pltpu.sync_copy(aggr_sh.at[pl.ds(TAIL_OFF, TAIL)],
                            out_hbm.at[cid].at[pl.ds(TAIL_OFF, TAIL)])

    return k(new_edge, receivers3, zeros_nd)


def _edge_mlp_body(gs_ref, gr_ref, ef_ref, wse_ref, wso_ref, wre_ref,
                   wro_ref, w1e_ref, b1_ref, w2_ref, b2_ref, g_ref, bg_ref,
                   ne_ref, eo_ref):
    ef = ef_ref[...]
    ws = gs_ref[...]
    wr = gr_ref[...]
    mask = jnp.int32(-65536)
    gse = jax.lax.bitcast_convert_type(ws << 16, jnp.float32)
    gso = jax.lax.bitcast_convert_type(ws & mask, jnp.float32)
    gre = jax.lax.bitcast_convert_type(wr << 16, jnp.float32)
    gro = jax.lax.bitcast_convert_type(wr & mask, jnp.float32)
    f32 = jnp.float32
    h = (jnp.dot(gse, wse_ref[0], preferred_element_type=f32)
         + jnp.dot(gso, wso_ref[0], preferred_element_type=f32)
         + jnp.dot(gre, wre_ref[0], preferred_element_type=f32)
         + jnp.dot(gro, wro_ref[0], preferred_element_type=f32)
         + jnp.dot(ef, w1e_ref[...], preferred_element_type=f32))
    h = jnp.maximum(h + b1_ref[...], 0.0)
    h = jnp.dot(h, w2_ref[...], preferred_element_type=f32)
    h = jnp.maximum(h + b2_ref[...], 0.0)
    mu = jnp.mean(h, axis=-1, keepdims=True)
    var = jnp.mean((h - mu) ** 2, axis=-1, keepdims=True)
    ne = (h - mu) / jnp.sqrt(var + 1e-5) * g_ref[...] + bg_ref[...]
    ne_ref[...] = ne
    eo_ref[...] = ef + ne


def _tc_edge_mlp(gs, gr, ef, Wse, Wso, Wre, Wro, W1e, be1, We2, be2, ge,
                 bge, E, D, BE=2000):
    NBh = (E // 2) // BE
    grid = (NBh, 2)
    pair = pl.BlockSpec((BE, D), lambda j, h: (j, 0))
    ehalf = pl.BlockSpec((BE, D), lambda j, h: (h * NBh + j, 0))
    wsel = pl.BlockSpec((1, D, D), lambda j, h: (h, 0, 0))
    full = lambda a: pl.BlockSpec(a.shape, lambda j, h: tuple(0 for _ in a.shape))
    return pl.pallas_call(
        _edge_mlp_body,
        grid=grid,
        in_specs=[pair, pair, ehalf, wsel, wsel, wsel, wsel, full(W1e),
                  full(be1), full(We2), full(be2), full(ge), full(bge)],
        out_specs=[ehalf, ehalf],
        out_shape=[jax.ShapeDtypeStruct((E, D), jnp.float32),
                   jax.ShapeDtypeStruct((E, D), jnp.float32)],
        compiler_params=pltpu.CompilerParams(
            dimension_semantics=("arbitrary", "arbitrary")),
    )(gs, gr, ef, Wse, Wso, Wre, Wro, W1e, be1, We2, be2, ge, bge)


def _node_mlp_body(nl_ref, a0_ref, a1_ref, w1_ref, b1_ref, w2_ref, b2_ref,
                   g_ref, bg_ref, out_ref):
    nl = nl_ref[...]
    aggr = a0_ref[...] + a1_ref[...]
    x = jnp.concatenate([nl, aggr], axis=-1)
    h = jnp.dot(x, w1_ref[...], preferred_element_type=jnp.float32)
    h = jnp.maximum(h + b1_ref[...], 0.0)
    h = jnp.dot(h, w2_ref[...], preferred_element_type=jnp.float32)
    h = jnp.maximum(h + b2_ref[...], 0.0)
    mu = jnp.mean(h, axis=-1, keepdims=True)
    var = jnp.mean((h - mu) ** 2, axis=-1, keepdims=True)
    nn = (h - mu) / jnp.sqrt(var + 1e-5) * g_ref[...] + bg_ref[...]
    out_ref[...] = nn + nl


def _tc_node_mlp(nl, aggr2, Wn1, bn1, Wn2, bn2, gn, bgn, N, D, BN=2000):
    grid = (N // BN,)
    blk = pl.BlockSpec((BN, D), lambda i: (i, 0))
    full = lambda a: pl.BlockSpec(a.shape, lambda i: tuple(0 for _ in a.shape))
    return pl.pallas_call(
        _node_mlp_body,
        grid=grid,
        in_specs=[blk, blk, blk, full(Wn1), full(bn1), full(Wn2), full(bn2),
                  full(gn), full(bgn)],
        out_specs=blk,
        out_shape=jax.ShapeDtypeStruct((N, D), jnp.float32),
        compiler_params=pltpu.CompilerParams(
            dimension_semantics=("arbitrary",)),
    )(nl, aggr2[0], aggr2[1], Wn1, bn1, Wn2, bn2, gn, bgn)


def kernel(node_latents, edge_features, senders, receivers, We1, be1, We2,
           be2, ge, bge, Wn1, bn1, Wn2, bn2, gn, bgn):
    B, N, D = node_latents.shape
    E = senders.shape[0]
    EPW = E // NW
    NCH = EPW // C

    nl = node_latents.reshape(N, D)
    ef = edge_features.reshape(E, D)
    s2 = senders.astype(jnp.int32).reshape(NW, EPW)
    r2 = receivers.astype(jnp.int32).reshape(NW, EPW)
    r3 = receivers.astype(jnp.int32).reshape(NW, NCH, C)

    nl32 = jax.lax.bitcast_convert_type(
        nl.astype(jnp.bfloat16).reshape(N, D // 2, 2), jnp.int32)
    gs, gr = _sc_gather(nl32, s2, r2, E, N, D // 2)
    W1s, W1r, W1e = We1[:D], We1[D:2 * D], We1[2 * D:]
    z = jnp.zeros((D // 2, D), jnp.float32)

    def _sel(Wrows):
        return jnp.stack([jnp.concatenate([Wrows, z], axis=0),
                          jnp.concatenate([z, Wrows], axis=0)])

    Wse, Wso = _sel(W1s[0::2]), _sel(W1s[1::2])
    Wre, Wro = _sel(W1r[0::2]), _sel(W1r[1::2])
    new_edge, edge_out = _tc_edge_mlp(
        gs, gr, ef, Wse, Wso, Wre, Wro, W1e, be1.reshape(1, D), We2,
        be2.reshape(1, D), ge.reshape(1, D), bge.reshape(1, D), E, D)
    zeros_nd = jnp.zeros((N, D), jnp.float32)
    aggr2 = _sc_scatter_add(new_edge, r3, zeros_nd, E, N, D)
    node_out = _tc_node_mlp(
        nl, aggr2, Wn1, bn1.reshape(1, D), Wn2, bn2.reshape(1, D),
        gn.reshape(1, D), bgn.reshape(1, D), N, D)
    return node_out.reshape(B, N, D), edge_out.reshape(B, E, D)


# speedup vs baseline: 1.4205x; 1.1584x over previous
import functools

import jax
import jax.numpy as jnp
from jax import lax
from jax.experimental import pallas as pl
from jax.experimental.pallas import tpu as pltpu
from jax.experimental.pallas import tpu_sc as plsc

NW = 32
NC = 2
NS = 16
C = 80


def _sc_gather(nl, senders2, receivers2, E, N, D):
    EPW = E // NW
    CG = 128
    NCH = EPW // CG
    TAIL = EPW - NCH * CG
    mesh = plsc.VectorSubcoreMesh(core_axis_name="c", subcore_axis_name="s")

    @functools.partial(
        pl.kernel,
        out_type=(jax.ShapeDtypeStruct((E // 2, 2 * D), jnp.int32),
                  jax.ShapeDtypeStruct((E // 2, 2 * D), jnp.int32)),
        mesh=mesh,
        scratch_types=[
            pltpu.VMEM((EPW,), jnp.int32),
            pltpu.VMEM((EPW,), jnp.int32),
            pltpu.VMEM((4, CG, D), jnp.int32),
            pltpu.VMEM((4, CG, D), jnp.int32),
            pltpu.SemaphoreType.DMA,
            pltpu.SemaphoreType.DMA,
            pltpu.SemaphoreType.DMA,
            pltpu.SemaphoreType.DMA,
            pltpu.SemaphoreType.DMA,
            pltpu.SemaphoreType.DMA,
            pltpu.SemaphoreType.DMA,
            pltpu.SemaphoreType.DMA,
        ],
        compiler_params=pltpu.CompilerParams(use_tc_tiling_on_sc=False),
    )
    def k(nl_hbm, s_hbm, r_hbm, gs_hbm, gr_hbm, sidx, ridx, srow, rrow,
          ss0, ss1, ss2, ss3, rs0, rs1, rs2, rs3):
        ssem = (ss0, ss1, ss2, ss3)
        rsem = (rs0, rs1, rs2, rs3)
        cid = lax.axis_index("c")
        sid = lax.axis_index("s")
        wid = sid * NC + cid
        base = wid * EPW
        first_half = wid < NW // 2
        rowbase = base - jnp.where(first_half, 0, E // 2)
        colbase = jnp.where(first_half, 0, D)
        pltpu.sync_copy(s_hbm.at[wid], sidx)
        pltpu.sync_copy(r_hbm.at[wid], ridx)

        def fire(i, b):
            pltpu.async_copy(nl_hbm.at[sidx.at[pl.ds(i * CG, CG)]],
                             srow.at[b], ssem[b])
            pltpu.async_copy(nl_hbm.at[ridx.at[pl.ds(i * CG, CG)]],
                             rrow.at[b], rsem[b])

        def wait_gather(i, b):
            pltpu.make_async_copy(nl_hbm.at[sidx.at[pl.ds(i * CG, CG)]],
                                  srow.at[b], ssem[b]).wait()
            pltpu.make_async_copy(nl_hbm.at[ridx.at[pl.ds(i * CG, CG)]],
                                  rrow.at[b], rsem[b]).wait()

        def fire_write(i, b):
            off = rowbase + i * CG
            pltpu.async_copy(srow.at[b],
                             gs_hbm.at[pl.ds(off, CG), pl.ds(colbase, D)],
                             ssem[b])
            pltpu.async_copy(rrow.at[b],
                             gr_hbm.at[pl.ds(off, CG), pl.ds(colbase, D)],
                             rsem[b])

        def wait_write(i, b):
            off = rowbase + i * CG
            pltpu.make_async_copy(srow.at[b],
                                  gs_hbm.at[pl.ds(off, CG),
                                            pl.ds(colbase, D)],
                                  ssem[b]).wait()
            pltpu.make_async_copy(rrow.at[b],
                                  gr_hbm.at[pl.ds(off, CG),
                                            pl.ds(colbase, D)],
                                  rsem[b]).wait()

        for b in range(4):
            fire(b, b)

        def body(m, carry):
            i = 4 * m
            for b in range(4):
                wait_gather(i + b, b)
                fire_write(i + b, b)
            for b in range(4):
                wait_write(i + b, b)

                @pl.when(i + b + 4 < NCH)
                def _():
                    fire(i + b + 4, b)

            return carry

        lax.fori_loop(0, NCH // 4, body, 0)

        rem = NCH % 4
        for b in range(rem):
            i = (NCH // 4) * 4 + b
            wait_gather(i, b)
            fire_write(i, b)
        for b in range(rem):
            i = (NCH // 4) * 4 + b
            wait_write(i, b)

        toff = NCH * CG
        pltpu.async_copy(nl_hbm.at[sidx.at[pl.ds(toff, TAIL)]],
                         srow.at[0, pl.ds(0, TAIL)], ss0)
        pltpu.async_copy(nl_hbm.at[ridx.at[pl.ds(toff, TAIL)]],
                         rrow.at[0, pl.ds(0, TAIL)], rs0)
        pltpu.make_async_copy(nl_hbm.at[sidx.at[pl.ds(toff, TAIL)]],
                              srow.at[0, pl.ds(0, TAIL)], ss0).wait()
        pltpu.make_async_copy(nl_hbm.at[ridx.at[pl.ds(toff, TAIL)]],
                              rrow.at[0, pl.ds(0, TAIL)], rs0).wait()
        pltpu.sync_copy(srow.at[0, pl.ds(0, TAIL)],
                        gs_hbm.at[pl.ds(rowbase + toff, TAIL),
                                  pl.ds(colbase, D)])
        pltpu.sync_copy(rrow.at[0, pl.ds(0, TAIL)],
                        gr_hbm.at[pl.ds(rowbase + toff, TAIL),
                                  pl.ds(colbase, D)])

    return k(nl, senders2, receivers2)


def _sc_scatter_add(new_edge, receivers3, zeros_nd, E, N, D):
    NCH = receivers3.shape[1]
    EPW = NCH * C
    SLAB = (N // NS) // 8 * 8
    TAIL_OFF = SLAB * NS
    TAIL = N - TAIL_OFF
    mesh = plsc.VectorSubcoreMesh(core_axis_name="c", subcore_axis_name="s")

    @functools.partial(
        pl.kernel,
        out_type=jax.ShapeDtypeStruct((NC, N, D), jnp.float32),
        mesh=mesh,
        scratch_types=[
            pltpu.VMEM((NCH, C), jnp.int32),
            pltpu.VMEM((2, C, D), jnp.float32),
            pltpu.VMEM_SHARED((N, D), jnp.float32),
            pltpu.SemaphoreType.DMA,
            pltpu.SemaphoreType.DMA,
        ],
    )
    def k(ne_hbm, r_hbm, z_hbm, out_hbm, ridx, rows, aggr_sh, lm0, lm1):
        lsem = (lm0, lm1)
        cid = lax.axis_index("c")
        sid = lax.axis_index("s")
        wid = sid * NC + cid
        base = wid * EPW
        pltpu.sync_copy(z_hbm.at[pl.ds(sid * SLAB, SLAB)],
                        aggr_sh.at[pl.ds(sid * SLAB, SLAB)])

        @pl.when(sid == 0)
        def _():
            pltpu.sync_copy(z_hbm.at[pl.ds(TAIL_OFF, TAIL)],
                            aggr_sh.at[pl.ds(TAIL_OFF, TAIL)])

        pltpu.sync_copy(r_hbm.at[wid], ridx)
        plsc.subcore_barrier()

        def fire_load(i, b):
            pltpu.async_copy(ne_hbm.at[pl.ds(base + i * C, C)], rows.at[b],
                             lsem[b])

        def wait_load(i, b):
            pltpu.make_async_copy(ne_hbm.at[pl.ds(base + i * C, C)],
                                  rows.at[b], lsem[b]).wait()

        fire_load(0, 0)
        fire_load(1, 1)

        def body(j, carry):
            i0 = 2 * j
            i1 = 2 * j + 1
            wait_load(i0, 0)
            pltpu.sync_copy(rows.at[0], aggr_sh.at[ridx.at[i0]], add=True)

            @pl.when(i0 + 2 < NCH)
            def _():
                fire_load(i0 + 2, 0)

            wait_load(i1, 1)
            pltpu.sync_copy(rows.at[1], aggr_sh.at[ridx.at[i1]], add=True)

            @pl.when(i1 + 2 < NCH)
            def _():
                fire_load(i1 + 2, 1)

            return carry

        lax.fori_loop(0, NCH // 2, body, 0)
        wait_load(NCH - 1, 0)
        pltpu.sync_copy(rows.at[0], aggr_sh.at[ridx.at[NCH - 1]], add=True)
        plsc.subcore_barrier()
        pltpu.sync_copy(aggr_sh.at[pl.ds(sid * SLAB, SLAB)],
                        out_hbm.at[cid].at[pl.ds(sid * SLAB, SLAB)])

        @pl.when(sid == 0)
        def _():
            pltpu.sync_copy(aggr_sh.at[pl.ds(TAIL_OFF, TAIL)],
                            out_hbm.at[cid].at[pl.ds(TAIL_OFF, TAIL)])

    return k(new_edge, receivers3, zeros_nd)


def _edge_mlp_body(gs_ref, gr_ref, ef_ref, wse_ref, wso_ref, wre_ref,
                   wro_ref, w1e_ref, b1_ref, w2_ref, b2_ref, g_ref, bg_ref,
                   ne_ref, eo_ref):
    ef = ef_ref[...]
    ws = gs_ref[...]
    wr = gr_ref[...]
    mask = jnp.int32(-65536)
    gse = jax.lax.bitcast_convert_type(ws << 16, jnp.float32)
    gso = jax.lax.bitcast_convert_type(ws & mask, jnp.float32)
    gre = jax.lax.bitcast_convert_type(wr << 16, jnp.float32)
    gro = jax.lax.bitcast_convert_type(wr & mask, jnp.float32)
    f32 = jnp.float32
    h = (jnp.dot(gse, wse_ref[0], preferred_element_type=f32)
         + jnp.dot(gso, wso_ref[0], preferred_element_type=f32)
         + jnp.dot(gre, wre_ref[0], preferred_element_type=f32)
         + jnp.dot(gro, wro_ref[0], preferred_element_type=f32)
         + jnp.dot(ef, w1e_ref[...], preferred_element_type=f32))
    h = jnp.maximum(h + b1_ref[...], 0.0)
    h = jnp.dot(h, w2_ref[...], preferred_element_type=f32)
    h = jnp.maximum(h + b2_ref[...], 0.0)
    mu = jnp.mean(h, axis=-1, keepdims=True)
    var = jnp.mean((h - mu) ** 2, axis=-1, keepdims=True)
    ne = (h - mu) / jnp.sqrt(var + 1e-5) * g_ref[...] + bg_ref[...]
    ne_ref[...] = ne
    eo_ref[...] = ef + ne


def _tc_edge_mlp(gs, gr, ef, Wse, Wso, Wre, Wro, W1e, be1, We2, be2, ge,
                 bge, E, D, BE=2000):
    NBh = (E // 2) // BE
    grid = (NBh, 2)
    pair = pl.BlockSpec((BE, D), lambda j, h: (j, 0))
    ehalf = pl.BlockSpec((BE, D), lambda j, h: (h * NBh + j, 0))
    wsel = pl.BlockSpec((1, D, D), lambda j, h: (h, 0, 0))
    full = lambda a: pl.BlockSpec(a.shape, lambda j, h: tuple(0 for _ in a.shape))
    return pl.pallas_call(
        _edge_mlp_body,
        grid=grid,
        in_specs=[pair, pair, ehalf, wsel, wsel, wsel, wsel, full(W1e),
                  full(be1), full(We2), full(be2), full(ge), full(bge)],
        out_specs=[ehalf, ehalf],
        out_shape=[jax.ShapeDtypeStruct((E, D), jnp.float32),
                   jax.ShapeDtypeStruct((E, D), jnp.float32)],
        compiler_params=pltpu.CompilerParams(
            dimension_semantics=("arbitrary", "arbitrary")),
    )(gs, gr, ef, Wse, Wso, Wre, Wro, W1e, be1, We2, be2, ge, bge)


def _node_mlp_body(nl_ref, a0_ref, a1_ref, w1_ref, b1_ref, w2_ref, b2_ref,
                   g_ref, bg_ref, out_ref):
    nl = nl_ref[...]
    aggr = a0_ref[...] + a1_ref[...]
    x = jnp.concatenate([nl, aggr], axis=-1)
    h = jnp.dot(x, w1_ref[...], preferred_element_type=jnp.float32)
    h = jnp.maximum(h + b1_ref[...], 0.0)
    h = jnp.dot(h, w2_ref[...], preferred_element_type=jnp.float32)
    h = jnp.maximum(h + b2_ref[...], 0.0)
    mu = jnp.mean(h, axis=-1, keepdims=True)
    var = jnp.mean((h - mu) ** 2, axis=-1, keepdims=True)
    nn = (h - mu) / jnp.sqrt(var + 1e-5) * g_ref[...] + bg_ref[...]
    out_ref[...] = nn + nl


def _tc_node_mlp(nl, aggr2, Wn1, bn1, Wn2, bn2, gn, bgn, N, D, BN=2000):
    grid = (N // BN,)
    blk = pl.BlockSpec((BN, D), lambda i: (i, 0))
    full = lambda a: pl.BlockSpec(a.shape, lambda i: tuple(0 for _ in a.shape))
    return pl.pallas_call(
        _node_mlp_body,
        grid=grid,
        in_specs=[blk, blk, blk, full(Wn1), full(bn1), full(Wn2), full(bn2),
                  full(gn), full(bgn)],
        out_specs=blk,
        out_shape=jax.ShapeDtypeStruct((N, D), jnp.float32),
        compiler_params=pltpu.CompilerParams(
            dimension_semantics=("arbitrary",)),
    )(nl, aggr2[0], aggr2[1], Wn1, bn1, Wn2, bn2, gn, bgn)


def kernel(node_latents, edge_features, senders, receivers, We1, be1, We2,
           be2, ge, bge, Wn1, bn1, Wn2, bn2, gn, bgn):
    B, N, D = node_latents.shape
    E = senders.shape[0]
    EPW = E // NW
    NCH = EPW // C

    nl = node_latents.reshape(N, D)
    ef = edge_features.reshape(E, D)
    s2 = senders.astype(jnp.int32).reshape(NW, EPW)
    r2 = receivers.astype(jnp.int32).reshape(NW, EPW)
    r3 = receivers.astype(jnp.int32).reshape(NW, NCH, C)

    nl32 = jax.lax.bitcast_convert_type(
        nl.astype(jnp.bfloat16).reshape(N, D // 2, 2), jnp.int32)
    gs, gr = _sc_gather(nl32, s2, r2, E, N, D // 2)
    W1s, W1r, W1e = We1[:D], We1[D:2 * D], We1[2 * D:]
    z = jnp.zeros((D // 2, D), jnp.float32)

    def _sel(Wrows):
        return jnp.stack([jnp.concatenate([Wrows, z], axis=0),
                          jnp.concatenate([z, Wrows], axis=0)])

    Wse, Wso = _sel(W1s[0::2]), _sel(W1s[1::2])
    Wre, Wro = _sel(W1r[0::2]), _sel(W1r[1::2])
    new_edge, edge_out = _tc_edge_mlp(
        gs, gr, ef, Wse, Wso, Wre, Wro, W1e, be1.reshape(1, D), We2,
        be2.reshape(1, D), ge.reshape(1, D), bge.reshape(1, D), E, D)
    zeros_nd = jnp.zeros((N, D), jnp.float32)
    aggr2 = _sc_scatter_add(new_edge, r3, zeros_nd, E, N, D)
    node_out = _tc_node_mlp(
        nl, aggr2, Wn1, bn1.reshape(1, D), Wn2, bn2.reshape(1, D),
        gn.reshape(1, D), bgn.reshape(1, D), N, D)
    return node_out.reshape(B, N, D), edge_out.reshape(B, E, D)


# speedup vs baseline: 1.4935x; 1.0514x over previous
import functools

import jax
import jax.numpy as jnp
from jax import lax
from jax.experimental import pallas as pl
from jax.experimental.pallas import tpu as pltpu
from jax.experimental.pallas import tpu_sc as plsc

NW = 32
NC = 2
NS = 16
C = 80


def _sc_gather(nl, senders2, receivers2, E, N, D):
    EPW = E // NW
    CG = 128
    NCH = EPW // CG
    TAIL = EPW - NCH * CG
    mesh = plsc.VectorSubcoreMesh(core_axis_name="c", subcore_axis_name="s")

    @functools.partial(
        pl.kernel,
        out_type=(jax.ShapeDtypeStruct((E // 2, 2 * D), jnp.int32),
                  jax.ShapeDtypeStruct((E // 2, 2 * D), jnp.int32)),
        mesh=mesh,
        scratch_types=[
            pltpu.VMEM((EPW,), jnp.int32),
            pltpu.VMEM((EPW,), jnp.int32),
            pltpu.VMEM((4, CG, D), jnp.int32),
            pltpu.VMEM((4, CG, D), jnp.int32),
            pltpu.SemaphoreType.DMA,
            pltpu.SemaphoreType.DMA,
            pltpu.SemaphoreType.DMA,
            pltpu.SemaphoreType.DMA,
            pltpu.SemaphoreType.DMA,
            pltpu.SemaphoreType.DMA,
            pltpu.SemaphoreType.DMA,
            pltpu.SemaphoreType.DMA,
        ],
        compiler_params=pltpu.CompilerParams(use_tc_tiling_on_sc=False),
    )
    def k(nl_hbm, s_hbm, r_hbm, gs_hbm, gr_hbm, sidx, ridx, srow, rrow,
          ss0, ss1, ss2, ss3, rs0, rs1, rs2, rs3):
        ssem = (ss0, ss1, ss2, ss3)
        rsem = (rs0, rs1, rs2, rs3)
        cid = lax.axis_index("c")
        sid = lax.axis_index("s")
        wid = sid * NC + cid
        base = wid * EPW
        first_half = wid < NW // 2
        rowbase = base - jnp.where(first_half, 0, E // 2)
        colbase = jnp.where(first_half, 0, D)
        pltpu.sync_copy(s_hbm.at[wid], sidx)
        pltpu.sync_copy(r_hbm.at[wid], ridx)

        def fire(i, b):
            pltpu.async_copy(nl_hbm.at[sidx.at[pl.ds(i * CG, CG)]],
                             srow.at[b], ssem[b])
            pltpu.async_copy(nl_hbm.at[ridx.at[pl.ds(i * CG, CG)]],
                             rrow.at[b], rsem[b])

        def wait_gather(i, b):
            pltpu.make_async_copy(nl_hbm.at[sidx.at[pl.ds(i * CG, CG)]],
                                  srow.at[b], ssem[b]).wait()
            pltpu.make_async_copy(nl_hbm.at[ridx.at[pl.ds(i * CG, CG)]],
                                  rrow.at[b], rsem[b]).wait()

        def fire_write(i, b):
            off = rowbase + i * CG
            pltpu.async_copy(srow.at[b],
                             gs_hbm.at[pl.ds(off, CG), pl.ds(colbase, D)],
                             ssem[b])
            pltpu.async_copy(rrow.at[b],
                             gr_hbm.at[pl.ds(off, CG), pl.ds(colbase, D)],
                             rsem[b])

        def wait_write(i, b):
            off = rowbase + i * CG
            pltpu.make_async_copy(srow.at[b],
                                  gs_hbm.at[pl.ds(off, CG),
                                            pl.ds(colbase, D)],
                                  ssem[b]).wait()
            pltpu.make_async_copy(rrow.at[b],
                                  gr_hbm.at[pl.ds(off, CG),
                                            pl.ds(colbase, D)],
                                  rsem[b]).wait()

        for b in range(4):
            fire(b, b)

        def body(m, carry):
            i = 4 * m
            for b in range(4):
                wait_gather(i + b, b)
                fire_write(i + b, b)
            for b in range(4):
                wait_write(i + b, b)

                @pl.when(i + b + 4 < NCH)
                def _():
                    fire(i + b + 4, b)

            return carry

        lax.fori_loop(0, NCH // 4, body, 0)

        rem = NCH % 4
        for b in range(rem):
            i = (NCH // 4) * 4 + b
            wait_gather(i, b)
            fire_write(i, b)
        for b in range(rem):
            i = (NCH // 4) * 4 + b
            wait_write(i, b)

        toff = NCH * CG
        pltpu.async_copy(nl_hbm.at[sidx.at[pl.ds(toff, TAIL)]],
                         srow.at[0, pl.ds(0, TAIL)], ss0)
        pltpu.async_copy(nl_hbm.at[ridx.at[pl.ds(toff, TAIL)]],
                         rrow.at[0, pl.ds(0, TAIL)], rs0)
        pltpu.make_async_copy(nl_hbm.at[sidx.at[pl.ds(toff, TAIL)]],
                              srow.at[0, pl.ds(0, TAIL)], ss0).wait()
        pltpu.make_async_copy(nl_hbm.at[ridx.at[pl.ds(toff, TAIL)]],
                              rrow.at[0, pl.ds(0, TAIL)], rs0).wait()
        pltpu.sync_copy(srow.at[0, pl.ds(0, TAIL)],
                        gs_hbm.at[pl.ds(rowbase + toff, TAIL),
                                  pl.ds(colbase, D)])
        pltpu.sync_copy(rrow.at[0, pl.ds(0, TAIL)],
                        gr_hbm.at[pl.ds(rowbase + toff, TAIL),
                                  pl.ds(colbase, D)])

    return k(nl, senders2, receivers2)


def _sc_scatter_add(new_edge, receivers3, zeros_nd, E, N, D, C, EOFF):
    NCH = receivers3.shape[1]
    EPW = NCH * C
    SLAB = (N // NS) // 8 * 8
    TAIL_OFF = SLAB * NS
    TAIL = N - TAIL_OFF
    mesh = plsc.VectorSubcoreMesh(core_axis_name="c", subcore_axis_name="s")

    @functools.partial(
        pl.kernel,
        out_type=jax.ShapeDtypeStruct((NC, N, D), jnp.float32),
        mesh=mesh,
        scratch_types=[
            pltpu.VMEM((NCH, C), jnp.int32),
            pltpu.VMEM((2, C, D), jnp.float32),
            pltpu.VMEM_SHARED((N, D), jnp.float32),
            pltpu.SemaphoreType.DMA,
            pltpu.SemaphoreType.DMA,
        ],
    )
    def k(ne_hbm, r_hbm, z_hbm, out_hbm, ridx, rows, aggr_sh, lm0, lm1):
        lsem = (lm0, lm1)
        cid = lax.axis_index("c")
        sid = lax.axis_index("s")
        wid = sid * NC + cid
        base = EOFF + wid * EPW
        pltpu.sync_copy(z_hbm.at[pl.ds(sid * SLAB, SLAB)],
                        aggr_sh.at[pl.ds(sid * SLAB, SLAB)])

        @pl.when(sid == 0)
        def _():
            pltpu.sync_copy(z_hbm.at[pl.ds(TAIL_OFF, TAIL)],
                            aggr_sh.at[pl.ds(TAIL_OFF, TAIL)])

        pltpu.sync_copy(r_hbm.at[wid], ridx)
        plsc.subcore_barrier()

        def fire_load(i, b):
            pltpu.async_copy(ne_hbm.at[pl.ds(base + i * C, C)], rows.at[b],
                             lsem[b])

        def wait_load(i, b):
            pltpu.make_async_copy(ne_hbm.at[pl.ds(base + i * C, C)],
                                  rows.at[b], lsem[b]).wait()

        fire_load(0, 0)
        fire_load(1, 1)

        def body(j, carry):
            i0 = 2 * j
            i1 = 2 * j + 1
            wait_load(i0, 0)
            pltpu.sync_copy(rows.at[0], aggr_sh.at[ridx.at[i0]], add=True)

            @pl.when(i0 + 2 < NCH)
            def _():
                fire_load(i0 + 2, 0)

            wait_load(i1, 1)
            pltpu.sync_copy(rows.at[1], aggr_sh.at[ridx.at[i1]], add=True)

            @pl.when(i1 + 2 < NCH)
            def _():
                fire_load(i1 + 2, 1)

            return carry

        lax.fori_loop(0, NCH // 2, body, 0)
        wait_load(NCH - 1, 0)
        pltpu.sync_copy(rows.at[0], aggr_sh.at[ridx.at[NCH - 1]], add=True)
        plsc.subcore_barrier()
        pltpu.sync_copy(aggr_sh.at[pl.ds(sid * SLAB, SLAB)],
                        out_hbm.at[cid].at[pl.ds(sid * SLAB, SLAB)])

        @pl.when(sid == 0)
        def _():
            pltpu.sync_copy(aggr_sh.at[pl.ds(TAIL_OFF, TAIL)],
                            out_hbm.at[cid].at[pl.ds(TAIL_OFF, TAIL)])

    return k(new_edge, receivers3, zeros_nd)


def _edge_mlp_body(gs_ref, gr_ref, ef_ref, wse_ref, wso_ref, wre_ref,
                   wro_ref, w1e_ref, b1_ref, w2_ref, b2_ref, g_ref, bg_ref,
                   ne_ref, eo_ref):
    ef = ef_ref[...]
    ws = gs_ref[...]
    wr = gr_ref[...]
    mask = jnp.int32(-65536)
    gse = jax.lax.bitcast_convert_type(ws << 16, jnp.float32)
    gso = jax.lax.bitcast_convert_type(ws & mask, jnp.float32)
    gre = jax.lax.bitcast_convert_type(wr << 16, jnp.float32)
    gro = jax.lax.bitcast_convert_type(wr & mask, jnp.float32)
    f32 = jnp.float32
    h = (jnp.dot(gse, wse_ref[0], preferred_element_type=f32)
         + jnp.dot(gso, wso_ref[0], preferred_element_type=f32)
         + jnp.dot(gre, wre_ref[0], preferred_element_type=f32)
         + jnp.dot(gro, wro_ref[0], preferred_element_type=f32)
         + jnp.dot(ef, w1e_ref[...], preferred_element_type=f32))
    h = jnp.maximum(h + b1_ref[...], 0.0)
    h = jnp.dot(h, w2_ref[...], preferred_element_type=f32)
    h = jnp.maximum(h + b2_ref[...], 0.0)
    mu = jnp.mean(h, axis=-1, keepdims=True)
    var = jnp.mean((h - mu) ** 2, axis=-1, keepdims=True)
    ne = (h - mu) / jnp.sqrt(var + 1e-5) * g_ref[...] + bg_ref[...]
    ne_ref[...] = ne
    eo_ref[...] = ef + ne


def _tc_edge_mlp(gs, gr, ef, Wse, Wso, Wre, Wro, W1e, be1, We2, be2, ge,
                 bge, E, D, OBLK, eo_prev=None, BE=2000):
    NBh = (E // 2) // BE
    grid = (NBh, 2)
    pair = pl.BlockSpec((BE, D), lambda j, h: (j, 0))
    nehalf = pl.BlockSpec((BE, D), lambda j, h: (h * NBh + j, 0))
    eohalf = pl.BlockSpec((BE, D), lambda j, h: (OBLK + h * NBh + j, 0))
    wsel = pl.BlockSpec((1, D, D), lambda j, h: (h, 0, 0))
    full = lambda a: pl.BlockSpec(a.shape, lambda j, h: tuple(0 for _ in a.shape))
    Efull = ef.shape[0]
    ins = [gs, gr, ef, Wse, Wso, Wre, Wro, W1e, be1, We2, be2, ge, bge]
    in_specs = [pair, pair, eohalf, wsel, wsel, wsel, wsel, full(W1e),
                full(be1), full(We2), full(be2), full(ge), full(bge)]
    aliases = {}
    body = _edge_mlp_body
    if eo_prev is not None:
        ins += [eo_prev]
        in_specs += [pl.BlockSpec(memory_space=pl.ANY)]
        aliases = {13: 1}
        body = _edge_mlp_body_aliased
    return pl.pallas_call(
        body,
        grid=grid,
        in_specs=in_specs,
        out_specs=[nehalf, eohalf],
        out_shape=[jax.ShapeDtypeStruct((E, D), jnp.float32),
                   jax.ShapeDtypeStruct((Efull, D), jnp.float32)],
        input_output_aliases=aliases,
        compiler_params=pltpu.CompilerParams(
            dimension_semantics=("arbitrary", "arbitrary")),
    )(*ins)


def _edge_mlp_body_aliased(gs_ref, gr_ref, ef_ref, wse_ref, wso_ref, wre_ref,
                           wro_ref, w1e_ref, b1_ref, w2_ref, b2_ref, g_ref,
                           bg_ref, eo_prev_ref, ne_ref, eo_ref):
    del eo_prev_ref
    _edge_mlp_body(gs_ref, gr_ref, ef_ref, wse_ref, wso_ref, wre_ref,
                   wro_ref, w1e_ref, b1_ref, w2_ref, b2_ref, g_ref, bg_ref,
                   ne_ref, eo_ref)


def _node_mlp_body(nl_ref, a0_ref, a1_ref, a2_ref, a3_ref, w1_ref, b1_ref,
                   w2_ref, b2_ref, g_ref, bg_ref, out_ref):
    nl = nl_ref[...]
    aggr = (a0_ref[...] + a1_ref[...]) + (a2_ref[...] + a3_ref[...])
    x = jnp.concatenate([nl, aggr], axis=-1)
    h = jnp.dot(x, w1_ref[...], preferred_element_type=jnp.float32)
    h = jnp.maximum(h + b1_ref[...], 0.0)
    h = jnp.dot(h, w2_ref[...], preferred_element_type=jnp.float32)
    h = jnp.maximum(h + b2_ref[...], 0.0)
    mu = jnp.mean(h, axis=-1, keepdims=True)
    var = jnp.mean((h - mu) ** 2, axis=-1, keepdims=True)
    nn = (h - mu) / jnp.sqrt(var + 1e-5) * g_ref[...] + bg_ref[...]
    out_ref[...] = nn + nl


def _tc_node_mlp(nl, aggrA, aggrB, Wn1, bn1, Wn2, bn2, gn, bgn, N, D,
                 BN=2000):
    grid = (N // BN,)
    blk = pl.BlockSpec((BN, D), lambda i: (i, 0))
    full = lambda a: pl.BlockSpec(a.shape, lambda i: tuple(0 for _ in a.shape))
    return pl.pallas_call(
        _node_mlp_body,
        grid=grid,
        in_specs=[blk, blk, blk, blk, blk, full(Wn1), full(bn1), full(Wn2),
                  full(bn2), full(gn), full(bgn)],
        out_specs=blk,
        out_shape=jax.ShapeDtypeStruct((N, D), jnp.float32),
        compiler_params=pltpu.CompilerParams(
            dimension_semantics=("arbitrary",)),
    )(nl, aggrA[0], aggrA[1], aggrB[0], aggrB[1], Wn1, bn1, Wn2, bn2, gn,
      bgn)


def kernel(node_latents, edge_features, senders, receivers, We1, be1, We2,
           be2, ge, bge, Wn1, bn1, Wn2, bn2, gn, bgn):
    B, N, D = node_latents.shape
    E = senders.shape[0]
    Eh = E // 2
    EPWh = Eh // NW
    CS = 40

    nl = node_latents.reshape(N, D)
    ef = edge_features.reshape(E, D)
    snd = senders.astype(jnp.int32)
    rcv = receivers.astype(jnp.int32)
    nl32 = jax.lax.bitcast_convert_type(
        nl.astype(jnp.bfloat16).reshape(N, D // 2, 2), jnp.int32)

    W1s, W1r, W1e = We1[:D], We1[D:2 * D], We1[2 * D:]
    z = jnp.zeros((D // 2, D), jnp.float32)

    def _sel(Wrows):
        return jnp.stack([jnp.concatenate([Wrows, z], axis=0),
                          jnp.concatenate([z, Wrows], axis=0)])

    Wse, Wso = _sel(W1s[0::2]), _sel(W1s[1::2])
    Wre, Wro = _sel(W1r[0::2]), _sel(W1r[1::2])
    b1e = be1.reshape(1, D)
    b2e = be2.reshape(1, D)
    g1 = ge.reshape(1, D)
    bg1 = bge.reshape(1, D)
    zeros_nd = jnp.zeros((N, D), jnp.float32)

    sA = snd[:Eh].reshape(NW, EPWh)
    rA = rcv[:Eh].reshape(NW, EPWh)
    sB = snd[Eh:].reshape(NW, EPWh)
    rB = rcv[Eh:].reshape(NW, EPWh)
    gsA, grA = _sc_gather(nl32, sA, rA, Eh, N, D // 2)
    gsB, grB = _sc_gather(nl32, sB, rB, Eh, N, D // 2)

    NBLK = 2000
    neA, eoA = _tc_edge_mlp(gsA, grA, ef, Wse, Wso, Wre, Wro, W1e, b1e, We2,
                            b2e, g1, bg1, Eh, D, 0)
    r3A = rcv[:Eh].reshape(NW, EPWh // CS, CS)
    aggrA = _sc_scatter_add(neA, r3A, zeros_nd, Eh, N, D, CS, 0)
    neB, eoB = _tc_edge_mlp(gsB, grB, ef, Wse, Wso, Wre, Wro, W1e, b1e, We2,
                            b2e, g1, bg1, Eh, D, Eh // NBLK, eoA)
    r3B = rcv[Eh:].reshape(NW, EPWh // CS, CS)
    aggrB = _sc_scatter_add(neB, r3B, zeros_nd, Eh, N, D, CS, 0)

    node_out = _tc_node_mlp(
        nl, aggrA, aggrB, Wn1, bn1.reshape(1, D), Wn2, bn2.reshape(1, D),
        gn.reshape(1, D), bgn.reshape(1, D), N, D)
    return node_out.reshape(B, N, D), eoB.reshape(B, E, D)
